# Initial kernel scaffold; baseline (speedup 1.0000x reference)
import jax
import jax.numpy as jnp
from jax.experimental import pallas as pl


def kernel(x, W):
    raise NotImplementedError("write your pallas kernel here")


# speedup vs baseline: 1.3129x; 1.3129x over previous
import functools

import jax
import jax.numpy as jnp
from jax import lax
from jax.experimental import pallas as pl
from jax.experimental.pallas import tpu as pltpu
from jax.experimental.pallas import tpu_sc as plsc

K = 8192
D = 256
N = 32 * 576
M_TILE = 256
M_TILES = N // M_TILE
LOSS_SCALE = 1.25 / (N * D)


def _w2_body(w_ref, w2_ref):
    w = w_ref[...]
    w2_ref[...] = jnp.sum(w * w, axis=1, keepdims=True)


_w2_call = pl.pallas_call(
    _w2_body,
    in_specs=[pl.BlockSpec((K, D), lambda: (0, 0))],
    out_specs=pl.BlockSpec((K, 1), lambda: (0, 0)),
    out_shape=jax.ShapeDtypeStruct((K, 1), jnp.float32),
)


def _dist_body(x_ref, wt_ref, w2_ref, idx_ref, loss_ref):
    pid = pl.program_id(0)

    wt = wt_ref[...]
    x = x_ref[...]
    m = lax.dot_general(x, wt, (((1,), (0,)), ((), ())),
                        preferred_element_type=jnp.float32)
    x2 = jnp.sum(x * x, axis=1, keepdims=True)
    d = x2 + w2_ref[...] - 2.0 * m

    dmin = jnp.min(d, axis=1, keepdims=True)
    iota = lax.broadcasted_iota(jnp.int32, (M_TILE, K), 1)
    idx = jnp.min(jnp.where(d == dmin, iota, jnp.int32(K)),
                  axis=1, keepdims=True)
    idx_ref[...] = idx

    part = jnp.sum(dmin)
    prev = jnp.where(pid == 0, 0.0, loss_ref[0, 0])
    acc = prev + part
    loss_ref[0, 0] = jnp.where(pid == M_TILES - 1, acc * LOSS_SCALE, acc)


_dist_call = pl.pallas_call(
    _dist_body,
    grid=(M_TILES,),
    in_specs=[
        pl.BlockSpec((M_TILE, D), lambda m: (m, 0)),
        pl.BlockSpec((D, K), lambda m: (0, 0)),
        pl.BlockSpec((1, K), lambda m: (0, 0)),
    ],
    out_specs=[
        pl.BlockSpec((M_TILE, 1), lambda m: (m, 0)),
        pl.BlockSpec(memory_space=pltpu.SMEM, block_shape=(1, 1),
                     index_map=lambda m: (0, 0)),
    ],
    out_shape=[
        jax.ShapeDtypeStruct((N, 1), jnp.int32),
        jax.ShapeDtypeStruct((1, 1), jnp.float32),
    ],
    compiler_params=pltpu.CompilerParams(
        dimension_semantics=("arbitrary",)),
)


SC_CORES = 2
SC_SUBCORES = 16


@functools.cache
def _make_sc_gather():
    nw = SC_CORES * SC_SUBCORES
    b_per_w = N // nw
    ch = b_per_w // 2
    mesh = plsc.VectorSubcoreMesh(core_axis_name="c", subcore_axis_name="s")

    @functools.partial(
        pl.kernel, mesh=mesh,
        out_type=jax.ShapeDtypeStruct((N, D), jnp.float32),
        scratch_types=[
            pltpu.VMEM((ch,), jnp.int32),
            pltpu.VMEM((ch, D), jnp.float32),
            pltpu.SemaphoreType.DMA,
        ],
    )
    def gather(table_hbm, idx_hbm, out_hbm, idx_v, rows_v, sem):
        wid = lax.axis_index("s") * SC_CORES + lax.axis_index("c")
        base = wid * b_per_w
        for c in range(b_per_w // ch):
            off = base + c * ch
            pltpu.sync_copy(idx_hbm.at[pl.ds(off, ch)], idx_v)
            pltpu.async_copy(table_hbm.at[idx_v], rows_v, sem).wait()
            pltpu.sync_copy(rows_v, out_hbm.at[pl.ds(off, ch)])

    return gather


def kernel(x, W):
    flat_x = x.reshape(N, D)
    w2row = _w2_call(W).reshape(1, K)
    idx2d, loss11 = _dist_call(flat_x, W.T, w2row)
    quant = _make_sc_gather()(W, idx2d.reshape(N))
    return (quant.reshape(x.shape), loss11.reshape(()), idx2d)


# speedup vs baseline: 1.3738x; 1.0463x over previous
import functools

import jax
import jax.numpy as jnp
from jax import lax
from jax.experimental import pallas as pl
from jax.experimental.pallas import tpu as pltpu
from jax.experimental.pallas import tpu_sc as plsc

K = 8192
D = 256
N = 32 * 576
M_TILE = 256
M_TILES = N // M_TILE
CHUNK = 256
LOSS_SCALE = 1.25 / (N * D)


HALF = N // 2
HALF_TILES = HALF // M_TILE


def _dist_body(x_ref, w_ref, iota_ref, idx_ref, loss_ref, wt_ref, w2_ref):
    pid = pl.program_id(0)

    @pl.when(pid == 0)
    def _():
        w = w_ref[...]
        wt_ref[...] = w.T
        w2_ref[...] = jnp.sum(w * w, axis=1, keepdims=True).T

    x = x_ref[...]
    x2 = jnp.sum(x * x, axis=1, keepdims=True)
    xd = x + x

    vm = None
    va = None
    for c in range(K // CHUNK):
        sl = pl.ds(c * CHUNK, CHUNK)
        m2 = lax.dot_general(xd, wt_ref[:, sl], (((1,), (0,)), ((), ())),
                             preferred_element_type=jnp.float32)
        d_c = x2 + w2_ref[:, sl] - m2
        if c == 0:
            vm = d_c
            va = jnp.zeros((M_TILE, CHUNK), jnp.float32)
        else:
            lt = d_c < vm
            va = jnp.where(lt, jnp.float32(c), va)
            vm = jnp.minimum(vm, d_c)

    dmin = jnp.min(vm, axis=1, keepdims=True)
    lane = jnp.broadcast_to(iota_ref[...], (M_TILE, CHUNK))
    abs_idx = va * jnp.float32(CHUNK) + lane
    idxf = jnp.min(jnp.where(vm == dmin, abs_idx, jnp.float32(K)),
                   axis=1, keepdims=True)
    idx_ref[...] = idxf.astype(jnp.int32)

    part = jnp.sum(dmin)
    prev = jnp.where(pid == 0, 0.0, loss_ref[0, 0])
    acc = prev + part
    loss_ref[0, 0] = jnp.where(pid == HALF_TILES - 1, acc * LOSS_SCALE, acc)


_dist_call = pl.pallas_call(
    _dist_body,
    grid=(HALF_TILES,),
    in_specs=[
        pl.BlockSpec((M_TILE, D), lambda m: (m, 0)),
        pl.BlockSpec((K, D), lambda m: (0, 0)),
        pl.BlockSpec((1, CHUNK), lambda m: (0, 0)),
    ],
    out_specs=[
        pl.BlockSpec((M_TILE, 1), lambda m: (m, 0)),
        pl.BlockSpec(memory_space=pltpu.SMEM, block_shape=(1, 1),
                     index_map=lambda m: (0, 0)),
    ],
    out_shape=[
        jax.ShapeDtypeStruct((HALF, 1), jnp.int32),
        jax.ShapeDtypeStruct((1, 1), jnp.float32),
    ],
    scratch_shapes=[pltpu.VMEM((D, K), jnp.float32),
                    pltpu.VMEM((1, K), jnp.float32)],
    compiler_params=pltpu.CompilerParams(
        dimension_semantics=("arbitrary",)),
)


SC_CORES = 2
SC_SUBCORES = 16


@functools.cache
def _make_sc_gather():
    nw = SC_CORES * SC_SUBCORES
    b_per_w = HALF // nw
    ch = b_per_w
    mesh = plsc.VectorSubcoreMesh(core_axis_name="c", subcore_axis_name="s")

    @functools.partial(
        pl.kernel, mesh=mesh,
        out_type=jax.ShapeDtypeStruct((HALF, D), jnp.float32),
        scratch_types=[
            pltpu.VMEM((ch,), jnp.int32),
            pltpu.VMEM((ch, D), jnp.float32),
            pltpu.SemaphoreType.DMA,
        ],
    )
    def gather(table_hbm, idx_hbm, out_hbm, idx_v, rows_v, sem):
        wid = lax.axis_index("s") * SC_CORES + lax.axis_index("c")
        base = wid * b_per_w
        for c in range(b_per_w // ch):
            off = base + c * ch
            pltpu.sync_copy(idx_hbm.at[pl.ds(off, ch)], idx_v)
            pltpu.async_copy(table_hbm.at[idx_v], rows_v, sem).wait()
            pltpu.sync_copy(rows_v, out_hbm.at[pl.ds(off, ch)])

    return gather


def kernel(x, W):
    flat_x = x.reshape(N, D)
    iota_row = jnp.arange(CHUNK, dtype=jnp.float32)[None, :]
    gather = _make_sc_gather()
    idx_a, loss_a = _dist_call(flat_x[:HALF], W, iota_row)
    quant_a = gather(W, idx_a.reshape(HALF))
    idx_b, loss_b = _dist_call(flat_x[HALF:], W, iota_row)
    quant_b = gather(W, idx_b.reshape(HALF))
    quant = jnp.concatenate([quant_a, quant_b], axis=0).reshape(x.shape)
    loss = loss_a[0, 0] + loss_b[0, 0]
    idx2d = jnp.concatenate([idx_a, idx_b], axis=0)
    return (quant, loss, idx2d)


# speedup vs baseline: 1.6045x; 1.1679x over previous
import functools

import jax
import jax.numpy as jnp
from jax import lax
from jax.experimental import pallas as pl
from jax.experimental.pallas import tpu as pltpu
from jax.experimental.pallas import tpu_sc as plsc

K = 8192
D = 256
N = 32 * 576
M_TILE = 256
M_TILES = N // M_TILE
CHUNK = 256
LOSS_SCALE = 1.25 / (N * D)


def _dist_body(x_ref, w_ref, iota_ref, idx_ref, loss_ref, wt_ref, w2_ref):
    pid = pl.program_id(0)

    @pl.when(pid == 0)
    def _():
        w = w_ref[...]
        wt_ref[...] = w.T
        w2_ref[...] = jnp.sum(w * w, axis=1, keepdims=True).T

    x = x_ref[...]
    x2 = jnp.sum(x * x, axis=1, keepdims=True)
    xd = x + x

    vm = None
    va = None
    for c in range(K // CHUNK):
        sl = pl.ds(c * CHUNK, CHUNK)
        m2 = lax.dot_general(xd, wt_ref[:, sl], (((1,), (0,)), ((), ())),
                             preferred_element_type=jnp.float32)
        d_c = x2 + w2_ref[:, sl] - m2
        if c == 0:
            vm = d_c
            va = jnp.zeros((M_TILE, CHUNK), jnp.float32)
        else:
            lt = d_c < vm
            va = jnp.where(lt, jnp.float32(c), va)
            vm = jnp.minimum(vm, d_c)

    dmin = jnp.min(vm, axis=1, keepdims=True)
    lane = jnp.broadcast_to(iota_ref[...], (M_TILE, CHUNK))
    abs_idx = va * jnp.float32(CHUNK) + lane
    idxf = jnp.min(jnp.where(vm == dmin, abs_idx, jnp.float32(K)),
                   axis=1, keepdims=True)
    idx_ref[...] = idxf.astype(jnp.int32)

    part = jnp.sum(dmin)
    prev = jnp.where(pid == 0, 0.0, loss_ref[0, 0])
    acc = prev + part
    loss_ref[0, 0] = jnp.where(pid == M_TILES - 1, acc * LOSS_SCALE, acc)


_dist_call = pl.pallas_call(
    _dist_body,
    grid=(M_TILES,),
    in_specs=[
        pl.BlockSpec((M_TILE, D), lambda m: (m, 0)),
        pl.BlockSpec((K, D), lambda m: (0, 0)),
        pl.BlockSpec((1, CHUNK), lambda m: (0, 0)),
    ],
    out_specs=[
        pl.BlockSpec((M_TILE, 1), lambda m: (m, 0)),
        pl.BlockSpec(memory_space=pltpu.SMEM, block_shape=(1, 1),
                     index_map=lambda m: (0, 0)),
    ],
    out_shape=[
        jax.ShapeDtypeStruct((N, 1), jnp.int32),
        jax.ShapeDtypeStruct((1, 1), jnp.float32),
    ],
    scratch_shapes=[pltpu.VMEM((D, K), jnp.float32),
                    pltpu.VMEM((1, K), jnp.float32)],
    compiler_params=pltpu.CompilerParams(
        dimension_semantics=("arbitrary",)),
)


SC_CORES = 2
SC_SUBCORES = 16


@functools.cache
def _make_sc_gather():
    nw = SC_CORES * SC_SUBCORES
    b_per_w = N // nw
    ch = b_per_w // 4
    mesh = plsc.VectorSubcoreMesh(core_axis_name="c", subcore_axis_name="s")

    @functools.partial(
        pl.kernel, mesh=mesh,
        out_type=jax.ShapeDtypeStruct((N, D), jnp.float32),
        scratch_types=[
            pltpu.VMEM((b_per_w,), jnp.int32),
            pltpu.VMEM((ch, D), jnp.float32),
            pltpu.VMEM((ch, D), jnp.float32),
            pltpu.SemaphoreType.DMA,
            pltpu.SemaphoreType.DMA,
        ],
    )
    def gather(table_hbm, idx_hbm, out_hbm, idx_v, rows0, rows1, sem0, sem1):
        wid = lax.axis_index("s") * SC_CORES + lax.axis_index("c")
        base = wid * b_per_w
        pltpu.sync_copy(idx_hbm.at[pl.ds(base, b_per_w)], idx_v)
        rows = (rows0, rows1)
        sems = (sem0, sem1)
        cps = [pltpu.async_copy(table_hbm.at[idx_v.at[pl.ds(0, ch)]],
                                rows[0], sems[0]), None]
        for c in range(b_per_w // ch):
            if c + 1 < b_per_w // ch:
                b = (c + 1) % 2
                cps[b] = pltpu.async_copy(
                    table_hbm.at[idx_v.at[pl.ds((c + 1) * ch, ch)]],
                    rows[b], sems[b])
            cps[c % 2].wait()
            pltpu.sync_copy(rows[c % 2], out_hbm.at[pl.ds(base + c * ch, ch)])

    return gather


def kernel(x, W):
    flat_x = x.reshape(N, D)
    iota_row = jnp.arange(CHUNK, dtype=jnp.float32)[None, :]
    idx2d, loss11 = _dist_call(flat_x, W, iota_row)
    quant = _make_sc_gather()(W, idx2d.reshape(N))
    return (quant.reshape(x.shape), loss11.reshape(()), idx2d)
